# baseline (speedup 1.0000x reference)
import functools

import jax
import jax.numpy as jnp
from jax.experimental import pallas as pl
from jax.experimental.pallas import tpu as pltpu

N, D, H, E, K, O = 8192, 2048, 512, 16, 2, 10
OP = 128
BN = 512


def _moe_body(xf_ref, xb_ref, gw_ref, gb_ref, w1_ref, b1_ref, w2_ref, b2_ref,
              probs_ref, out_ref):
    e = pl.program_id(1)

    logits = jnp.dot(xf_ref[...], gw_ref[...],
                     preferred_element_type=jnp.float32) + gb_ref[...]
    m = jnp.max(logits, axis=-1, keepdims=True)
    ex = jnp.exp(logits - m)
    probs = ex / jnp.sum(ex, axis=-1, keepdims=True)

    @pl.when(e == 0)
    def _():
        probs_ref[...] = probs

    lane = jax.lax.broadcasted_iota(jnp.int32, probs.shape, 1)
    m1 = jnp.max(probs, axis=-1, keepdims=True)
    i1 = jnp.min(jnp.where(probs >= m1, lane, E), axis=-1, keepdims=True)
    p2 = jnp.where(lane == i1, -jnp.inf, probs)
    m2 = jnp.max(p2, axis=-1, keepdims=True)
    i2 = jnp.min(jnp.where(p2 >= m2, lane, E), axis=-1, keepdims=True)
    s = m1 + m2
    we = jnp.where(i1 == e, m1 / s, 0.0) + jnp.where(i2 == e, m2 / s, 0.0)

    h = jnp.dot(xb_ref[...], w1_ref[0],
                preferred_element_type=jnp.float32) + b1_ref[0]
    h = jnp.maximum(h, 0.0).astype(jnp.bfloat16)
    y = jnp.dot(h, w2_ref[0],
                preferred_element_type=jnp.float32) + b2_ref[0]

    @pl.when(e == 0)
    def _():
        out_ref[...] = jnp.zeros_like(out_ref)

    out_ref[...] += we * y


@jax.jit
def _moe(x, gate_W, gate_b, W1, b1, W2, b2):
    xb = x.astype(jnp.bfloat16)
    w1b = W1.astype(jnp.bfloat16)
    w2b = jnp.zeros((E, H, OP), jnp.bfloat16).at[:, :, :O].set(
        W2.astype(jnp.bfloat16))
    b1r = b1.reshape(E, 1, H)
    b2p = jnp.zeros((E, 1, OP), jnp.float32).at[:, 0, :O].set(b2)

    grid = (N // BN, E)
    probs, out = pl.pallas_call(
        _moe_body,
        grid=grid,
        in_specs=[
            pl.BlockSpec((BN, D), lambda nb, e: (nb, 0)),
            pl.BlockSpec((BN, D), lambda nb, e: (nb, 0)),
            pl.BlockSpec((D, E), lambda nb, e: (0, 0)),
            pl.BlockSpec((E,), lambda nb, e: (0,)),
            pl.BlockSpec((1, D, H), lambda nb, e: (e, 0, 0)),
            pl.BlockSpec((1, 1, H), lambda nb, e: (e, 0, 0)),
            pl.BlockSpec((1, H, OP), lambda nb, e: (e, 0, 0)),
            pl.BlockSpec((1, 1, OP), lambda nb, e: (e, 0, 0)),
        ],
        out_specs=[
            pl.BlockSpec((BN, E), lambda nb, e: (nb, 0)),
            pl.BlockSpec((BN, OP), lambda nb, e: (nb, 0)),
        ],
        out_shape=[
            jax.ShapeDtypeStruct((N, E), jnp.float32),
            jax.ShapeDtypeStruct((N, OP), jnp.float32),
        ],
        compiler_params=pltpu.CompilerParams(
            dimension_semantics=("arbitrary", "arbitrary"),
        ),
    )(x, xb, gate_W, gate_b, w1b, b1r, w2b, b2p)
    return out[:, :O], probs


def kernel(x, gate_W, gate_b, W1, b1, W2, b2):
    return _moe(x, gate_W, gate_b, W1, b1, W2, b2)


# speedup vs baseline: 1.1053x; 1.1053x over previous
import functools

import jax
import jax.numpy as jnp
from jax.experimental import pallas as pl
from jax.experimental.pallas import tpu as pltpu

N, D, H, E, K, O = 8192, 2048, 512, 16, 2, 10
OP = 128
BN = 1024


def _moe_body(xf_ref, xb_ref, gw_ref, gb_ref, w1_ref, b1_ref, w2_ref, b2_ref,
              probs_ref, out_ref):
    e = pl.program_id(1)

    logits = jnp.dot(xf_ref[...], gw_ref[...],
                     preferred_element_type=jnp.float32) + gb_ref[...]
    m = jnp.max(logits, axis=-1, keepdims=True)
    ex = jnp.exp(logits - m)
    probs = ex / jnp.sum(ex, axis=-1, keepdims=True)

    @pl.when(e == 0)
    def _():
        probs_ref[...] = probs

    lane = jax.lax.broadcasted_iota(jnp.int32, probs.shape, 1)
    m1 = jnp.max(probs, axis=-1, keepdims=True)
    i1 = jnp.min(jnp.where(probs >= m1, lane, E), axis=-1, keepdims=True)
    p2 = jnp.where(lane == i1, -jnp.inf, probs)
    m2 = jnp.max(p2, axis=-1, keepdims=True)
    i2 = jnp.min(jnp.where(p2 >= m2, lane, E), axis=-1, keepdims=True)
    s = m1 + m2
    we = jnp.where(i1 == e, m1 / s, 0.0) + jnp.where(i2 == e, m2 / s, 0.0)

    h = jnp.dot(xb_ref[...], w1_ref[0],
                preferred_element_type=jnp.float32) + b1_ref[0]
    h = jnp.maximum(h, 0.0).astype(jnp.bfloat16)
    y = jnp.dot(h, w2_ref[0],
                preferred_element_type=jnp.float32) + b2_ref[0]

    @pl.when(e == 0)
    def _():
        out_ref[...] = jnp.zeros_like(out_ref)

    out_ref[...] += we * y


@jax.jit
def _moe(x, gate_W, gate_b, W1, b1, W2, b2):
    xb = x.astype(jnp.bfloat16)
    w1b = W1.astype(jnp.bfloat16)
    w2b = jnp.zeros((E, H, OP), jnp.bfloat16).at[:, :, :O].set(
        W2.astype(jnp.bfloat16))
    b1r = b1.reshape(E, 1, H)
    b2p = jnp.zeros((E, 1, OP), jnp.float32).at[:, 0, :O].set(b2)

    grid = (N // BN, E)
    probs, out = pl.pallas_call(
        _moe_body,
        grid=grid,
        in_specs=[
            pl.BlockSpec((BN, D), lambda nb, e: (nb, 0)),
            pl.BlockSpec((BN, D), lambda nb, e: (nb, 0)),
            pl.BlockSpec((D, E), lambda nb, e: (0, 0)),
            pl.BlockSpec((E,), lambda nb, e: (0,)),
            pl.BlockSpec((1, D, H), lambda nb, e: (e, 0, 0)),
            pl.BlockSpec((1, 1, H), lambda nb, e: (e, 0, 0)),
            pl.BlockSpec((1, H, OP), lambda nb, e: (e, 0, 0)),
            pl.BlockSpec((1, 1, OP), lambda nb, e: (e, 0, 0)),
        ],
        out_specs=[
            pl.BlockSpec((BN, E), lambda nb, e: (nb, 0)),
            pl.BlockSpec((BN, OP), lambda nb, e: (nb, 0)),
        ],
        out_shape=[
            jax.ShapeDtypeStruct((N, E), jnp.float32),
            jax.ShapeDtypeStruct((N, OP), jnp.float32),
        ],
        compiler_params=pltpu.CompilerParams(
            dimension_semantics=("arbitrary", "arbitrary"),
        ),
    )(x, xb, gate_W, gate_b, w1b, b1r, w2b, b2p)
    return out[:, :O], probs


def kernel(x, gate_W, gate_b, W1, b1, W2, b2):
    return _moe(x, gate_W, gate_b, W1, b1, W2, b2)


# speedup vs baseline: 1.6299x; 1.4745x over previous
import functools

import jax
import jax.numpy as jnp
from jax import lax
from jax.experimental import pallas as pl
from jax.experimental.pallas import tpu as pltpu
from jax.experimental.pallas import tpu_sc as plsc

N, D, H, E, K, O = 8192, 2048, 512, 16, 2, 10
OP = 128
BM = 128
S = N * K + E * BM
NBS = S // BM
BNG = 1024
NBG = N // BNG

NC, NS, L = 2, 16, 16
NW = NC * NS
TW = N // NW
CS = 16
NCH = TW // CS


def _gate_body(x_ref, gw_ref, gb_ref, probs_ref, r_ref, ss_ref, be_ref,
               cnt_ref):
    nb = pl.program_id(0)

    logits = jnp.dot(x_ref[...], gw_ref[...],
                     preferred_element_type=jnp.float32) + gb_ref[...]
    m = jnp.max(logits, axis=-1, keepdims=True)
    ex = jnp.exp(logits - m)
    probs = ex / jnp.sum(ex, axis=-1, keepdims=True)
    probs_ref[...] = probs

    lane = lax.broadcasted_iota(jnp.int32, probs.shape, 1)
    m1 = jnp.max(probs, axis=-1, keepdims=True)
    i1 = jnp.min(jnp.where(probs >= m1, lane, E), axis=-1, keepdims=True)
    p2 = jnp.where(lane == i1, -jnp.inf, probs)
    m2 = jnp.max(p2, axis=-1, keepdims=True)
    i2 = jnp.min(jnp.where(p2 >= m2, lane, E), axis=-1, keepdims=True)
    s = m1 + m2
    w1 = m1 / s
    w2 = m2 / s

    oh1 = (lane == i1).astype(jnp.float32)
    oh2 = (lane == i2).astype(jnp.float32)
    ssum = oh1 + oh2

    ri = lax.broadcasted_iota(jnp.int32, (BNG, BNG), 0)
    ci = lax.broadcasted_iota(jnp.int32, (BNG, BNG), 1)
    ltri = (ri > ci).astype(jnp.bfloat16)
    excl = jnp.dot(ltri, ssum.astype(jnp.bfloat16),
                   preferred_element_type=jnp.float32)

    @pl.when(nb == 0)
    def _():
        cnt_ref[...] = jnp.zeros_like(cnt_ref)

    base = cnt_ref[0:1, 0:E]
    g0 = excl + base
    r0 = jnp.sum(g0 * oh1, axis=-1, keepdims=True)
    r1 = jnp.sum(g0 * oh2, axis=-1, keepdims=True)

    blockcnt = jnp.sum(ssum, axis=0, keepdims=True)
    cnt_ref[0:1, 0:E] = base + blockcnt

    r_ref[...] = jnp.concatenate(
        [i1.astype(jnp.float32), i2.astype(jnp.float32), r0, r1, w1, w2,
         jnp.zeros_like(w1), jnp.zeros_like(w1)], axis=1)

    @pl.when(nb == NBG - 1)
    def _():
        total = base + blockcnt
        padded = jnp.floor((total + (BM - 1)) / BM) * BM
        ea = lax.broadcasted_iota(jnp.int32, (E, E), 0)
        eb = lax.broadcasted_iota(jnp.int32, (E, E), 1)
        ut = (ea < eb).astype(jnp.float32)
        start = jnp.dot(padded, ut,
                        preferred_element_type=jnp.float32)
        start_p = jnp.concatenate(
            [start, jnp.zeros((1, 128 - E), jnp.float32)], axis=1)
        ss_ref[...] = start_p

        blk = lax.broadcasted_iota(jnp.int32, (1, 256), 1) * BM
        acc = jnp.full((1, 256), -1, jnp.int32)
        for e in range(E):
            acc = acc + jnp.where(blk >= start[0, e].astype(jnp.int32), 1, 0)
        be_ref[...] = acc


def _gate_route(x, gate_W, gate_b):
    return pl.pallas_call(
        _gate_body,
        grid=(NBG,),
        in_specs=[
            pl.BlockSpec((BNG, D), lambda nb: (nb, 0)),
            pl.BlockSpec((D, E), lambda nb: (0, 0)),
            pl.BlockSpec((E,), lambda nb: (0,)),
        ],
        out_specs=[
            pl.BlockSpec((BNG, E), lambda nb: (nb, 0)),
            pl.BlockSpec((BNG, 8), lambda nb: (nb, 0)),
            pl.BlockSpec((1, 128), lambda nb: (0, 0)),
            pl.BlockSpec((1, 256), lambda nb: (0, 0)),
        ],
        out_shape=[
            jax.ShapeDtypeStruct((N, E), jnp.float32),
            jax.ShapeDtypeStruct((N, 8), jnp.float32),
            jax.ShapeDtypeStruct((1, 128), jnp.float32),
            jax.ShapeDtypeStruct((1, 256), jnp.int32),
        ],
        scratch_shapes=[pltpu.VMEM((8, 128), jnp.float32)],
        compiler_params=pltpu.CompilerParams(
            dimension_semantics=("arbitrary",),
        ),
    )(x, gate_W, gate_b)


def _pos_chunk(rv, ssv, t0):
    tok = (lax.iota(jnp.int32, L) + t0) * 8
    i1 = plsc.load_gather(rv, [tok])
    i2 = plsc.load_gather(rv, [tok + 1])
    r0 = plsc.load_gather(rv, [tok + 2])
    r1 = plsc.load_gather(rv, [tok + 3])
    s0 = plsc.load_gather(ssv, [i1.astype(jnp.int32)])
    s1 = plsc.load_gather(ssv, [i2.astype(jnp.int32)])
    pos0 = (s0 + r0).astype(jnp.int32)
    pos1 = (s1 + r1).astype(jnp.int32)
    return pos0, pos1


def _dispatch(r, ss, xb3):
    mesh = plsc.VectorSubcoreMesh(core_axis_name="c", subcore_axis_name="s")

    @functools.partial(
        pl.kernel, mesh=mesh,
        compiler_params=pltpu.CompilerParams(needs_layout_passes=False),
        out_type=jax.ShapeDtypeStruct((S, D // 128, 128), jnp.float32),
        scratch_types=[
            pltpu.VMEM((TW * 8,), jnp.float32),
            pltpu.VMEM((128,), jnp.float32),
            pltpu.VMEM((CS, D // 128, 128), jnp.float32),
            pltpu.VMEM((CS, D // 128, 128), jnp.float32),
            pltpu.VMEM((CS, D // 128, 128), jnp.float32),
            pltpu.VMEM((CS,), jnp.int32),
            pltpu.VMEM((CS,), jnp.int32),
            pltpu.VMEM((CS,), jnp.int32),
            pltpu.VMEM((CS,), jnp.int32),
            pltpu.VMEM((CS,), jnp.int32),
            pltpu.VMEM((CS,), jnp.int32),
            pltpu.SemaphoreType.DMA,
            pltpu.SemaphoreType.DMA,
            pltpu.SemaphoreType.DMA,
            pltpu.SemaphoreType.DMA,
            pltpu.SemaphoreType.DMA,
            pltpu.SemaphoreType.DMA,
        ],
    )
    def k(r_hbm, ss_hbm, x_hbm, xs_hbm, rv, ssv, xb0, xb1, xb2, ia0, ia1,
          ia2, ib0, ib1, ib2, gs0, gs1, gs2, ws0, ws1, ws2):
        wid = lax.axis_index("s") * NC + lax.axis_index("c")
        base = wid * TW
        pltpu.sync_copy(r_hbm.at[pl.ds(base * 8, TW * 8)], rv)
        pltpu.sync_copy(ss_hbm.at[0], ssv)

        xbufs = (xb0, xb1, xb2)
        idx0 = (ia0, ia1, ia2)
        idx1 = (ib0, ib1, ib2)
        gsem = (gs0, gs1, gs2)
        wsem = (ws0, ws1, ws2)

        gh = [None, None, None]
        sh = [None] * NCH
        for c in (0, 1):
            gh[c] = pltpu.async_copy(
                x_hbm.at[pl.ds(base + c * CS, CS)], xbufs[c], gsem[c])
        for c in range(NCH):
            b = c % 3
            if c >= 1:
                pb = (c - 1) % 3
                sh[c - 1][0].wait()
                sh[c - 1][1].wait()
            if c + 2 < NCH:
                nb2 = (c + 2) % 3
                gh[nb2] = pltpu.async_copy(
                    x_hbm.at[pl.ds(base + (c + 2) * CS, CS)],
                    xbufs[nb2], gsem[nb2])
            gh[b].wait()
            for k2 in range(CS // L):
                p0, p1 = _pos_chunk(rv, ssv, c * CS + k2 * L)
                idx0[b][pl.ds(k2 * L, L)] = p0
                idx1[b][pl.ds(k2 * L, L)] = p1
            sh[c] = (
                pltpu.async_copy(xbufs[b], xs_hbm.at[idx0[b]], wsem[b]),
                pltpu.async_copy(xbufs[b], xs_hbm.at[idx1[b]], wsem[b]),
            )
        sh[NCH - 1][0].wait()
        sh[NCH - 1][1].wait()

    return k(r, ss, xb3)


def _mlp_body(be_ref, xs_ref, w1_ref, b1_ref, w2_ref, b2_ref, y_ref):
    h = jnp.dot(xs_ref[...].astype(jnp.bfloat16), w1_ref[0],
                preferred_element_type=jnp.float32) + b1_ref[0]
    h = jnp.maximum(h, 0.0).astype(jnp.bfloat16)
    y_ref[...] = jnp.dot(h, w2_ref[0],
                         preferred_element_type=jnp.float32) + b2_ref[0]


def _grouped_mlp(be, xs2, w1b, b1r, w2p, b2p):
    grid_spec = pltpu.PrefetchScalarGridSpec(
        num_scalar_prefetch=1,
        grid=(NBS,),
        in_specs=[
            pl.BlockSpec((BM, D), lambda i, be: (i, 0)),
            pl.BlockSpec((1, D, H), lambda i, be: (be[i], 0, 0)),
            pl.BlockSpec((1, 1, H), lambda i, be: (be[i], 0, 0)),
            pl.BlockSpec((1, H, OP), lambda i, be: (be[i], 0, 0)),
            pl.BlockSpec((1, 1, OP), lambda i, be: (be[i], 0, 0)),
        ],
        out_specs=pl.BlockSpec((BM, OP), lambda i, be: (i, 0)),
    )
    return pl.pallas_call(
        _mlp_body,
        grid_spec=grid_spec,
        out_shape=jax.ShapeDtypeStruct((S, OP), jnp.float32),
        compiler_params=pltpu.CompilerParams(
            dimension_semantics=("arbitrary",),
        ),
    )(be, xs2, w1b, b1r, w2p, b2p)


def _combine(r, ss, y):
    mesh = plsc.VectorSubcoreMesh(core_axis_name="c", subcore_axis_name="s")

    @functools.partial(
        pl.kernel, mesh=mesh,
        compiler_params=pltpu.CompilerParams(needs_layout_passes=False),
        out_type=jax.ShapeDtypeStruct((N, OP), jnp.float32),
        scratch_types=[
            pltpu.VMEM((TW * 8,), jnp.float32),
            pltpu.VMEM((128,), jnp.float32),
            pltpu.VMEM((TW,), jnp.int32),
            pltpu.VMEM((TW,), jnp.int32),
            pltpu.VMEM((TW, OP), jnp.float32),
            pltpu.VMEM((TW, OP), jnp.float32),
            pltpu.VMEM((TW, OP), jnp.float32),
            pltpu.SemaphoreType.DMA,
            pltpu.SemaphoreType.DMA,
        ],
    )
    def k(r_hbm, ss_hbm, y_hbm, out_hbm, rv, ssv, ix0, ix1, y0, y1, ob,
          sem0, sem1):
        wid = lax.axis_index("s") * NC + lax.axis_index("c")
        base = wid * TW
        pltpu.sync_copy(r_hbm.at[pl.ds(base * 8, TW * 8)], rv)
        pltpu.sync_copy(ss_hbm.at[0], ssv)
        for k2 in range(TW // L):
            p0, p1 = _pos_chunk(rv, ssv, k2 * L)
            ix0[pl.ds(k2 * L, L)] = p0
            ix1[pl.ds(k2 * L, L)] = p1
        h0 = pltpu.async_copy(y_hbm.at[ix0], y0, sem0)
        h1 = pltpu.async_copy(y_hbm.at[ix1], y1, sem1)
        h0.wait()
        h1.wait()
        for t in range(TW):
            wv1 = plsc.load_gather(rv, [jnp.full((L,), t * 8 + 4, jnp.int32)])
            wv2 = plsc.load_gather(rv, [jnp.full((L,), t * 8 + 5, jnp.int32)])
            ob[t, pl.ds(0, L)] = (wv1 * y0[t, pl.ds(0, L)]
                                  + wv2 * y1[t, pl.ds(0, L)])
        pltpu.sync_copy(ob, out_hbm.at[pl.ds(base, TW)])

    return k(r, ss, y)


@jax.jit
def _moe(x, gate_W, gate_b, W1, b1, W2, b2):
    probs, r2d, ss, be2d = _gate_route(x, gate_W, gate_b)
    r = r2d.reshape(N * 8)
    be = be2d.reshape(-1)[:NBS]

    x3 = x.reshape(N, D // 128, 128)
    xs3 = _dispatch(r, ss, x3)
    xs2 = xs3.reshape(S, D)

    w1b = W1.astype(jnp.bfloat16)
    b1r = b1.reshape(E, 1, H)
    w2p = jnp.zeros((E, H, OP), jnp.bfloat16).at[:, :, :O].set(
        W2.astype(jnp.bfloat16))
    b2p = jnp.zeros((E, 1, OP), jnp.float32).at[:, 0, :O].set(b2)
    y = _grouped_mlp(be, xs2, w1b, b1r, w2p, b2p)

    out16 = _combine(r, ss, y)
    return out16[:, :O], probs


def kernel(x, gate_W, gate_b, W1, b1, W2, b2):
    return _moe(x, gate_W, gate_b, W1, b1, W2, b2)


# speedup vs baseline: 2.3025x; 1.4127x over previous
import functools

import jax
import jax.numpy as jnp
from jax import lax
from jax.experimental import pallas as pl
from jax.experimental.pallas import tpu as pltpu
from jax.experimental.pallas import tpu_sc as plsc

N, D, H, E, K, O = 8192, 2048, 512, 16, 2, 10
OP = 128
BM = 128
S = N * K + E * BM
NBS = S // BM
BNG = 1024
NBG = N // BNG

NC, NS, L = 2, 16, 16
NW = NC * NS
TW = N // NW
CS = 16
NCH = TW // CS


def _gate_body(x_ref, gw_ref, gb_ref, probs_ref, r_ref, ss_ref, be_ref,
               cnt_ref):
    nb = pl.program_id(0)

    logits = jnp.dot(x_ref[...], gw_ref[...],
                     preferred_element_type=jnp.float32) + gb_ref[...]
    m = jnp.max(logits, axis=-1, keepdims=True)
    ex = jnp.exp(logits - m)
    probs = ex / jnp.sum(ex, axis=-1, keepdims=True)
    probs_ref[...] = probs

    lane = lax.broadcasted_iota(jnp.int32, probs.shape, 1)
    m1 = jnp.max(probs, axis=-1, keepdims=True)
    i1 = jnp.min(jnp.where(probs >= m1, lane, E), axis=-1, keepdims=True)
    p2 = jnp.where(lane == i1, -jnp.inf, probs)
    m2 = jnp.max(p2, axis=-1, keepdims=True)
    i2 = jnp.min(jnp.where(p2 >= m2, lane, E), axis=-1, keepdims=True)
    s = m1 + m2
    w1 = m1 / s
    w2 = m2 / s

    oh1 = (lane == i1).astype(jnp.float32)
    oh2 = (lane == i2).astype(jnp.float32)
    ssum = oh1 + oh2

    ri = lax.broadcasted_iota(jnp.int32, (BNG, BNG), 0)
    ci = lax.broadcasted_iota(jnp.int32, (BNG, BNG), 1)
    ltri = (ri > ci).astype(jnp.bfloat16)
    excl = jnp.dot(ltri, ssum.astype(jnp.bfloat16),
                   preferred_element_type=jnp.float32)

    @pl.when(nb == 0)
    def _():
        cnt_ref[...] = jnp.zeros_like(cnt_ref)

    base = cnt_ref[0:1, 0:E]
    g0 = excl + base
    r0 = jnp.sum(g0 * oh1, axis=-1, keepdims=True)
    r1 = jnp.sum(g0 * oh2, axis=-1, keepdims=True)

    blockcnt = jnp.sum(ssum, axis=0, keepdims=True)
    cnt_ref[0:1, 0:E] = base + blockcnt

    r_ref[...] = jnp.concatenate(
        [i1.astype(jnp.float32), i2.astype(jnp.float32), r0, r1, w1, w2,
         jnp.zeros_like(w1), jnp.zeros_like(w1)], axis=1)

    @pl.when(nb == NBG - 1)
    def _():
        total = base + blockcnt
        padded = jnp.floor((total + (BM - 1)) / BM) * BM
        ea = lax.broadcasted_iota(jnp.int32, (E, E), 0)
        eb = lax.broadcasted_iota(jnp.int32, (E, E), 1)
        ut = (ea < eb).astype(jnp.float32)
        start = jnp.dot(padded, ut,
                        preferred_element_type=jnp.float32)
        start_p = jnp.concatenate(
            [start, jnp.zeros((1, 128 - E), jnp.float32)], axis=1)
        ss_ref[...] = start_p

        blk = lax.broadcasted_iota(jnp.int32, (1, 256), 1) * BM
        acc = jnp.full((1, 256), -1, jnp.int32)
        for e in range(E):
            acc = acc + jnp.where(blk >= start[0, e].astype(jnp.int32), 1, 0)
        be_ref[...] = acc


def _gate_route(x, gate_W, gate_b):
    return pl.pallas_call(
        _gate_body,
        grid=(NBG,),
        in_specs=[
            pl.BlockSpec((BNG, D), lambda nb: (nb, 0)),
            pl.BlockSpec((D, E), lambda nb: (0, 0)),
            pl.BlockSpec((E,), lambda nb: (0,)),
        ],
        out_specs=[
            pl.BlockSpec((BNG, E), lambda nb: (nb, 0)),
            pl.BlockSpec((BNG, 8), lambda nb: (nb, 0)),
            pl.BlockSpec((1, 128), lambda nb: (0, 0)),
            pl.BlockSpec((1, 256), lambda nb: (0, 0)),
        ],
        out_shape=[
            jax.ShapeDtypeStruct((N, E), jnp.float32),
            jax.ShapeDtypeStruct((N, 8), jnp.float32),
            jax.ShapeDtypeStruct((1, 128), jnp.float32),
            jax.ShapeDtypeStruct((1, 256), jnp.int32),
        ],
        scratch_shapes=[pltpu.VMEM((8, 128), jnp.float32)],
        compiler_params=pltpu.CompilerParams(
            dimension_semantics=("arbitrary",),
        ),
    )(x, gate_W, gate_b)


def _pos_chunk(rv, ssv, t0):
    tok = (lax.iota(jnp.int32, L) + t0) * 8
    i1 = plsc.load_gather(rv, [tok])
    i2 = plsc.load_gather(rv, [tok + 1])
    r0 = plsc.load_gather(rv, [tok + 2])
    r1 = plsc.load_gather(rv, [tok + 3])
    s0 = plsc.load_gather(ssv, [i1.astype(jnp.int32)])
    s1 = plsc.load_gather(ssv, [i2.astype(jnp.int32)])
    pos0 = (s0 + r0).astype(jnp.int32)
    pos1 = (s1 + r1).astype(jnp.int32)
    return pos0, pos1


def _dispatch(r, ss, xb3):
    mesh = plsc.VectorSubcoreMesh(core_axis_name="c", subcore_axis_name="s")

    @functools.partial(
        pl.kernel, mesh=mesh,
        compiler_params=pltpu.CompilerParams(needs_layout_passes=False),
        out_type=jax.ShapeDtypeStruct((S, D), jnp.float32),
        scratch_types=[
            pltpu.VMEM((TW * 8,), jnp.float32),
            pltpu.VMEM((128,), jnp.float32),
            pltpu.VMEM((CS, D), jnp.float32),
            pltpu.VMEM((CS, D), jnp.float32),
            pltpu.VMEM((CS, D), jnp.float32),
            pltpu.VMEM((CS,), jnp.int32),
            pltpu.VMEM((CS,), jnp.int32),
            pltpu.VMEM((CS,), jnp.int32),
            pltpu.VMEM((CS,), jnp.int32),
            pltpu.VMEM((CS,), jnp.int32),
            pltpu.VMEM((CS,), jnp.int32),
            pltpu.SemaphoreType.DMA,
            pltpu.SemaphoreType.DMA,
            pltpu.SemaphoreType.DMA,
            pltpu.SemaphoreType.DMA,
            pltpu.SemaphoreType.DMA,
            pltpu.SemaphoreType.DMA,
        ],
    )
    def k(r_hbm, ss_hbm, x_hbm, xs_hbm, rv, ssv, xb0, xb1, xb2, ia0, ia1,
          ia2, ib0, ib1, ib2, gs0, gs1, gs2, ws0, ws1, ws2):
        wid = lax.axis_index("s") * NC + lax.axis_index("c")
        base = wid * TW
        pltpu.sync_copy(r_hbm.at[pl.ds(base * 8, TW * 8)], rv)
        pltpu.sync_copy(ss_hbm.at[0], ssv)

        xbufs = (xb0, xb1, xb2)
        idx0 = (ia0, ia1, ia2)
        idx1 = (ib0, ib1, ib2)
        gsem = (gs0, gs1, gs2)
        wsem = (ws0, ws1, ws2)

        gh = [None, None, None]
        sh = [None] * NCH
        for c in (0, 1):
            gh[c] = pltpu.async_copy(
                x_hbm.at[pl.ds(base + c * CS, CS)], xbufs[c], gsem[c])
        for c in range(NCH):
            b = c % 3
            if c >= 1:
                pb = (c - 1) % 3
                sh[c - 1][0].wait()
                sh[c - 1][1].wait()
            if c + 2 < NCH:
                nb2 = (c + 2) % 3
                gh[nb2] = pltpu.async_copy(
                    x_hbm.at[pl.ds(base + (c + 2) * CS, CS)],
                    xbufs[nb2], gsem[nb2])
            gh[b].wait()
            for k2 in range(CS // L):
                p0, p1 = _pos_chunk(rv, ssv, c * CS + k2 * L)
                idx0[b][pl.ds(k2 * L, L)] = p0
                idx1[b][pl.ds(k2 * L, L)] = p1
            sh[c] = (
                pltpu.async_copy(xbufs[b], xs_hbm.at[idx0[b]], wsem[b]),
                pltpu.async_copy(xbufs[b], xs_hbm.at[idx1[b]], wsem[b]),
            )
        sh[NCH - 1][0].wait()
        sh[NCH - 1][1].wait()

    return k(r, ss, xb3)


def _mlp_body(be_ref, xs_ref, w1_ref, b1_ref, w2_ref, b2_ref, y_ref):
    h = jnp.dot(xs_ref[...].astype(jnp.bfloat16), w1_ref[0],
                preferred_element_type=jnp.float32) + b1_ref[0]
    h = jnp.maximum(h, 0.0).astype(jnp.bfloat16)
    y_ref[...] = jnp.dot(h, w2_ref[0],
                         preferred_element_type=jnp.float32) + b2_ref[0]


def _grouped_mlp(be, xs2, w1b, b1r, w2p, b2p):
    grid_spec = pltpu.PrefetchScalarGridSpec(
        num_scalar_prefetch=1,
        grid=(NBS,),
        in_specs=[
            pl.BlockSpec((BM, D), lambda i, be: (i, 0)),
            pl.BlockSpec((1, D, H), lambda i, be: (be[i], 0, 0)),
            pl.BlockSpec((1, 1, H), lambda i, be: (be[i], 0, 0)),
            pl.BlockSpec((1, H, OP), lambda i, be: (be[i], 0, 0)),
            pl.BlockSpec((1, 1, OP), lambda i, be: (be[i], 0, 0)),
        ],
        out_specs=pl.BlockSpec((BM, OP), lambda i, be: (i, 0)),
    )
    return pl.pallas_call(
        _mlp_body,
        grid_spec=grid_spec,
        out_shape=jax.ShapeDtypeStruct((S, OP), jnp.float32),
        compiler_params=pltpu.CompilerParams(
            dimension_semantics=("arbitrary",),
        ),
    )(be, xs2, w1b, b1r, w2p, b2p)


def _combine(r, ss, y):
    mesh = plsc.VectorSubcoreMesh(core_axis_name="c", subcore_axis_name="s")

    @functools.partial(
        pl.kernel, mesh=mesh,
        compiler_params=pltpu.CompilerParams(needs_layout_passes=False),
        out_type=jax.ShapeDtypeStruct((N, OP), jnp.float32),
        scratch_types=[
            pltpu.VMEM((TW * 8,), jnp.float32),
            pltpu.VMEM((128,), jnp.float32),
            pltpu.VMEM((TW,), jnp.int32),
            pltpu.VMEM((TW,), jnp.int32),
            pltpu.VMEM((TW, OP), jnp.float32),
            pltpu.VMEM((TW, OP), jnp.float32),
            pltpu.VMEM((TW, OP), jnp.float32),
            pltpu.SemaphoreType.DMA,
            pltpu.SemaphoreType.DMA,
        ],
    )
    def k(r_hbm, ss_hbm, y_hbm, out_hbm, rv, ssv, ix0, ix1, y0, y1, ob,
          sem0, sem1):
        wid = lax.axis_index("s") * NC + lax.axis_index("c")
        base = wid * TW
        pltpu.sync_copy(r_hbm.at[pl.ds(base * 8, TW * 8)], rv)
        pltpu.sync_copy(ss_hbm.at[0], ssv)
        for k2 in range(TW // L):
            p0, p1 = _pos_chunk(rv, ssv, k2 * L)
            ix0[pl.ds(k2 * L, L)] = p0
            ix1[pl.ds(k2 * L, L)] = p1
        h0 = pltpu.async_copy(y_hbm.at[ix0], y0, sem0)
        h1 = pltpu.async_copy(y_hbm.at[ix1], y1, sem1)
        h0.wait()
        h1.wait()
        for t in range(TW):
            wv1 = plsc.load_gather(rv, [jnp.full((L,), t * 8 + 4, jnp.int32)])
            wv2 = plsc.load_gather(rv, [jnp.full((L,), t * 8 + 5, jnp.int32)])
            ob[t, pl.ds(0, L)] = (wv1 * y0[t, pl.ds(0, L)]
                                  + wv2 * y1[t, pl.ds(0, L)])
        pltpu.sync_copy(ob, out_hbm.at[pl.ds(base, TW)])

    return k(r, ss, y)


@jax.jit
def _moe(x, gate_W, gate_b, W1, b1, W2, b2):
    probs, r2d, ss, be2d = _gate_route(x, gate_W, gate_b)
    r = r2d.reshape(N * 8)
    be = be2d.reshape(-1)[:NBS]

    xs2 = _dispatch(r, ss, x)

    w1b = W1.astype(jnp.bfloat16)
    b1r = b1.reshape(E, 1, H)
    w2p = jnp.zeros((E, H, OP), jnp.bfloat16).at[:, :, :O].set(
        W2.astype(jnp.bfloat16))
    b2p = jnp.zeros((E, 1, OP), jnp.float32).at[:, 0, :O].set(b2)
    y = _grouped_mlp(be, xs2, w1b, b1r, w2p, b2p)

    out16 = _combine(r, ss, y)
    return out16[:, :O], probs


def kernel(x, gate_W, gate_b, W1, b1, W2, b2):
    return _moe(x, gate_W, gate_b, W1, b1, W2, b2)


# speedup vs baseline: 2.7296x; 1.1855x over previous
import functools

import jax
import jax.numpy as jnp
from jax import lax
from jax.experimental import pallas as pl
from jax.experimental.pallas import tpu as pltpu
from jax.experimental.pallas import tpu_sc as plsc

N, D, H, E, K, O = 8192, 2048, 512, 16, 2, 10
OP = 128
BM = 256
S = N * K + E * BM
NBS = S // BM
BNG = 1024
NBG = N // BNG

NC, NS, L = 2, 16, 16
NW = NC * NS
TW = N // NW
CS = 16
NCH = TW // CS


def _gate_body(x_ref, gw_ref, gb_ref, probs_ref, r_ref, ss_ref, be_ref,
               cnt_ref):
    nb = pl.program_id(0)

    logits = jnp.dot(x_ref[...], gw_ref[...],
                     preferred_element_type=jnp.float32) + gb_ref[...]
    m = jnp.max(logits, axis=-1, keepdims=True)
    ex = jnp.exp(logits - m)
    probs = ex / jnp.sum(ex, axis=-1, keepdims=True)
    probs_ref[...] = probs

    lane = lax.broadcasted_iota(jnp.int32, probs.shape, 1)
    m1 = jnp.max(probs, axis=-1, keepdims=True)
    i1 = jnp.min(jnp.where(probs >= m1, lane, E), axis=-1, keepdims=True)
    p2 = jnp.where(lane == i1, -jnp.inf, probs)
    m2 = jnp.max(p2, axis=-1, keepdims=True)
    i2 = jnp.min(jnp.where(p2 >= m2, lane, E), axis=-1, keepdims=True)
    s = m1 + m2
    w1 = m1 / s
    w2 = m2 / s

    oh1 = (lane == i1).astype(jnp.float32)
    oh2 = (lane == i2).astype(jnp.float32)
    ssum = oh1 + oh2

    ri = lax.broadcasted_iota(jnp.int32, (BNG, BNG), 0)
    ci = lax.broadcasted_iota(jnp.int32, (BNG, BNG), 1)
    ltri = (ri > ci).astype(jnp.bfloat16)
    excl = jnp.dot(ltri, ssum.astype(jnp.bfloat16),
                   preferred_element_type=jnp.float32)

    @pl.when(nb == 0)
    def _():
        cnt_ref[...] = jnp.zeros_like(cnt_ref)

    base = cnt_ref[0:1, 0:E]
    g0 = excl + base
    r0 = jnp.sum(g0 * oh1, axis=-1, keepdims=True)
    r1 = jnp.sum(g0 * oh2, axis=-1, keepdims=True)

    blockcnt = jnp.sum(ssum, axis=0, keepdims=True)
    cnt_ref[0:1, 0:E] = base + blockcnt

    r_ref[...] = jnp.concatenate(
        [i1.astype(jnp.float32), i2.astype(jnp.float32), r0, r1, w1, w2,
         jnp.zeros_like(w1), jnp.zeros_like(w1)], axis=1)

    @pl.when(nb == NBG - 1)
    def _():
        total = base + blockcnt
        padded = jnp.floor((total + (BM - 1)) / BM) * BM
        ea = lax.broadcasted_iota(jnp.int32, (E, E), 0)
        eb = lax.broadcasted_iota(jnp.int32, (E, E), 1)
        ut = (ea < eb).astype(jnp.float32)
        start = jnp.dot(padded, ut,
                        preferred_element_type=jnp.float32)
        start_p = jnp.concatenate(
            [start, jnp.zeros((1, 128 - E), jnp.float32)], axis=1)
        ss_ref[...] = start_p

        blk = lax.broadcasted_iota(jnp.int32, (1, 256), 1) * BM
        acc = jnp.full((1, 256), -1, jnp.int32)
        for e in range(E):
            acc = acc + jnp.where(blk >= start[0, e].astype(jnp.int32), 1, 0)
        be_ref[...] = acc


def _gate_route(x, gate_W, gate_b):
    return pl.pallas_call(
        _gate_body,
        grid=(NBG,),
        in_specs=[
            pl.BlockSpec((BNG, D), lambda nb: (nb, 0)),
            pl.BlockSpec((D, E), lambda nb: (0, 0)),
            pl.BlockSpec((E,), lambda nb: (0,)),
        ],
        out_specs=[
            pl.BlockSpec((BNG, E), lambda nb: (nb, 0)),
            pl.BlockSpec((BNG, 8), lambda nb: (nb, 0)),
            pl.BlockSpec((1, 128), lambda nb: (0, 0)),
            pl.BlockSpec((1, 256), lambda nb: (0, 0)),
        ],
        out_shape=[
            jax.ShapeDtypeStruct((N, E), jnp.float32),
            jax.ShapeDtypeStruct((N, 8), jnp.float32),
            jax.ShapeDtypeStruct((1, 128), jnp.float32),
            jax.ShapeDtypeStruct((1, 256), jnp.int32),
        ],
        scratch_shapes=[pltpu.VMEM((8, 128), jnp.float32)],
        compiler_params=pltpu.CompilerParams(
            dimension_semantics=("arbitrary",),
        ),
    )(x, gate_W, gate_b)


def _pos_chunk(rv, ssv, t0):
    tok = (lax.iota(jnp.int32, L) + t0) * 8
    i1 = plsc.load_gather(rv, [tok])
    i2 = plsc.load_gather(rv, [tok + 1])
    r0 = plsc.load_gather(rv, [tok + 2])
    r1 = plsc.load_gather(rv, [tok + 3])
    s0 = plsc.load_gather(ssv, [i1.astype(jnp.int32)])
    s1 = plsc.load_gather(ssv, [i2.astype(jnp.int32)])
    pos0 = (s0 + r0).astype(jnp.int32)
    pos1 = (s1 + r1).astype(jnp.int32)
    return pos0, pos1


def _dispatch(r, ss, xb3):
    mesh = plsc.VectorSubcoreMesh(core_axis_name="c", subcore_axis_name="s")

    @functools.partial(
        pl.kernel, mesh=mesh,
        compiler_params=pltpu.CompilerParams(needs_layout_passes=False),
        out_type=jax.ShapeDtypeStruct((S, D), jnp.float32),
        scratch_types=[
            pltpu.VMEM((TW * 8,), jnp.float32),
            pltpu.VMEM((128,), jnp.float32),
            pltpu.VMEM((CS, D), jnp.float32),
            pltpu.VMEM((CS, D), jnp.float32),
            pltpu.VMEM((CS, D), jnp.float32),
            pltpu.VMEM((CS,), jnp.int32),
            pltpu.VMEM((CS,), jnp.int32),
            pltpu.VMEM((CS,), jnp.int32),
            pltpu.VMEM((CS,), jnp.int32),
            pltpu.VMEM((CS,), jnp.int32),
            pltpu.VMEM((CS,), jnp.int32),
            pltpu.SemaphoreType.DMA,
            pltpu.SemaphoreType.DMA,
            pltpu.SemaphoreType.DMA,
            pltpu.SemaphoreType.DMA,
            pltpu.SemaphoreType.DMA,
            pltpu.SemaphoreType.DMA,
        ],
    )
    def k(r_hbm, ss_hbm, x_hbm, xs_hbm, rv, ssv, xb0, xb1, xb2, ia0, ia1,
          ia2, ib0, ib1, ib2, gs0, gs1, gs2, ws0, ws1, ws2):
        wid = lax.axis_index("s") * NC + lax.axis_index("c")
        base = wid * TW
        pltpu.sync_copy(r_hbm.at[pl.ds(base * 8, TW * 8)], rv)
        pltpu.sync_copy(ss_hbm.at[0], ssv)

        xbufs = (xb0, xb1, xb2)
        idx0 = (ia0, ia1, ia2)
        idx1 = (ib0, ib1, ib2)
        gsem = (gs0, gs1, gs2)
        wsem = (ws0, ws1, ws2)

        gh = [None, None, None]
        sh = [None] * NCH
        for c in (0, 1):
            gh[c] = pltpu.async_copy(
                x_hbm.at[pl.ds(base + c * CS, CS)], xbufs[c], gsem[c])
        for c in range(NCH):
            b = c % 3
            if c >= 1:
                pb = (c - 1) % 3
                sh[c - 1][0].wait()
                sh[c - 1][1].wait()
            if c + 2 < NCH:
                nb2 = (c + 2) % 3
                gh[nb2] = pltpu.async_copy(
                    x_hbm.at[pl.ds(base + (c + 2) * CS, CS)],
                    xbufs[nb2], gsem[nb2])
            gh[b].wait()
            for k2 in range(CS // L):
                p0, p1 = _pos_chunk(rv, ssv, c * CS + k2 * L)
                idx0[b][pl.ds(k2 * L, L)] = p0
                idx1[b][pl.ds(k2 * L, L)] = p1
            sh[c] = (
                pltpu.async_copy(xbufs[b], xs_hbm.at[idx0[b]], wsem[b]),
                pltpu.async_copy(xbufs[b], xs_hbm.at[idx1[b]], wsem[b]),
            )
        sh[NCH - 1][0].wait()
        sh[NCH - 1][1].wait()

    return k(r, ss, xb3)


def _mlp_body(be_ref, xs_ref, w1_ref, b1_ref, w2_ref, b2_ref, y_ref,
              w1c_ref):
    i = pl.program_id(0)
    changed = jnp.logical_or(
        i == 0, be_ref[i] != be_ref[jnp.maximum(i - 1, 0)])

    @pl.when(changed)
    def _():
        w1c_ref[...] = w1_ref[0].astype(jnp.bfloat16)

    h = jnp.dot(xs_ref[...].astype(jnp.bfloat16), w1c_ref[...],
                preferred_element_type=jnp.float32) + b1_ref[0]
    h = jnp.maximum(h, 0.0).astype(jnp.bfloat16)
    y_ref[...] = jnp.dot(h, w2_ref[0],
                         preferred_element_type=jnp.float32) + b2_ref[0]


def _grouped_mlp(be, xs2, w1b, b1r, w2p, b2p):
    grid_spec = pltpu.PrefetchScalarGridSpec(
        num_scalar_prefetch=1,
        grid=(NBS,),
        in_specs=[
            pl.BlockSpec((BM, D), lambda i, be: (i, 0)),
            pl.BlockSpec((1, D, H), lambda i, be: (be[i], 0, 0)),
            pl.BlockSpec((1, 1, H), lambda i, be: (be[i], 0, 0)),
            pl.BlockSpec((1, H, OP), lambda i, be: (be[i], 0, 0)),
            pl.BlockSpec((1, 1, OP), lambda i, be: (be[i], 0, 0)),
        ],
        out_specs=pl.BlockSpec((BM, OP), lambda i, be: (i, 0)),
        scratch_shapes=[pltpu.VMEM((D, H), jnp.bfloat16)],
    )
    return pl.pallas_call(
        _mlp_body,
        grid_spec=grid_spec,
        out_shape=jax.ShapeDtypeStruct((S, OP), jnp.float32),
        compiler_params=pltpu.CompilerParams(
            dimension_semantics=("arbitrary",),
        ),
    )(be, xs2, w1b, b1r, w2p, b2p)


def _combine(r, ss, y):
    mesh = plsc.VectorSubcoreMesh(core_axis_name="c", subcore_axis_name="s")

    @functools.partial(
        pl.kernel, mesh=mesh,
        compiler_params=pltpu.CompilerParams(needs_layout_passes=False),
        out_type=jax.ShapeDtypeStruct((N, OP), jnp.float32),
        scratch_types=[
            pltpu.VMEM((TW * 8,), jnp.float32),
            pltpu.VMEM((128,), jnp.float32),
            pltpu.VMEM((TW,), jnp.int32),
            pltpu.VMEM((TW,), jnp.int32),
            pltpu.VMEM((TW, OP), jnp.float32),
            pltpu.VMEM((TW, OP), jnp.float32),
            pltpu.VMEM((TW, OP), jnp.float32),
            pltpu.SemaphoreType.DMA,
            pltpu.SemaphoreType.DMA,
        ],
    )
    def k(r_hbm, ss_hbm, y_hbm, out_hbm, rv, ssv, ix0, ix1, y0, y1, ob,
          sem0, sem1):
        wid = lax.axis_index("s") * NC + lax.axis_index("c")
        base = wid * TW
        pltpu.sync_copy(r_hbm.at[pl.ds(base * 8, TW * 8)], rv)
        pltpu.sync_copy(ss_hbm.at[0], ssv)
        for k2 in range(TW // L):
            p0, p1 = _pos_chunk(rv, ssv, k2 * L)
            ix0[pl.ds(k2 * L, L)] = p0
            ix1[pl.ds(k2 * L, L)] = p1
        h0 = pltpu.async_copy(y_hbm.at[ix0], y0, sem0)
        h1 = pltpu.async_copy(y_hbm.at[ix1], y1, sem1)
        h0.wait()
        h1.wait()
        for t in range(TW):
            wv1 = plsc.load_gather(rv, [jnp.full((L,), t * 8 + 4, jnp.int32)])
            wv2 = plsc.load_gather(rv, [jnp.full((L,), t * 8 + 5, jnp.int32)])
            ob[t, pl.ds(0, L)] = (wv1 * y0[t, pl.ds(0, L)]
                                  + wv2 * y1[t, pl.ds(0, L)])
        pltpu.sync_copy(ob, out_hbm.at[pl.ds(base, TW)])

    return k(r, ss, y)


@jax.jit
def _moe(x, gate_W, gate_b, W1, b1, W2, b2):
    probs, r2d, ss, be2d = _gate_route(x, gate_W, gate_b)
    r = r2d.reshape(N * 8)
    be = be2d.reshape(-1)[:NBS]

    xs2 = _dispatch(r, ss, x)

    b1r = b1.reshape(E, 1, H)
    w2p = jnp.zeros((E, H, OP), jnp.bfloat16).at[:, :, :O].set(
        W2.astype(jnp.bfloat16))
    b2p = jnp.zeros((E, 1, OP), jnp.float32).at[:, 0, :O].set(b2)
    y = _grouped_mlp(be, xs2, W1, b1r, w2p, b2p)

    out16 = _combine(r, ss, y)
    return out16[:, :O], probs


def kernel(x, gate_W, gate_b, W1, b1, W2, b2):
    return _moe(x, gate_W, gate_b, W1, b1, W2, b2)


# speedup vs baseline: 3.1021x; 1.1365x over previous
import functools

import jax
import jax.numpy as jnp
from jax import lax
from jax.experimental import pallas as pl
from jax.experimental.pallas import tpu as pltpu
from jax.experimental.pallas import tpu_sc as plsc

N, D, H, E, K, O = 8192, 2048, 512, 16, 2, 10
OP = 128
BM = 256
S = N * K + E * BM
NBS = S // BM
BNG = 1024
NBG = N // BNG

NC, NS, L = 2, 16, 16
NW = NC * NS
TW = N // NW
CS = 16
NCH = TW // CS


def _gate_body(x_ref, gw_ref, gb_ref, probs_ref, r_ref, ss_ref, be_ref,
               cnt_ref):
    nb = pl.program_id(0)

    logits = jnp.dot(x_ref[...], gw_ref[...],
                     preferred_element_type=jnp.float32) + gb_ref[...]
    m = jnp.max(logits, axis=-1, keepdims=True)
    ex = jnp.exp(logits - m)
    probs = ex / jnp.sum(ex, axis=-1, keepdims=True)
    probs_ref[...] = probs

    lane = lax.broadcasted_iota(jnp.int32, probs.shape, 1)
    m1 = jnp.max(probs, axis=-1, keepdims=True)
    i1 = jnp.min(jnp.where(probs >= m1, lane, E), axis=-1, keepdims=True)
    p2 = jnp.where(lane == i1, -jnp.inf, probs)
    m2 = jnp.max(p2, axis=-1, keepdims=True)
    i2 = jnp.min(jnp.where(p2 >= m2, lane, E), axis=-1, keepdims=True)
    s = m1 + m2
    w1 = m1 / s
    w2 = m2 / s

    oh1 = (lane == i1).astype(jnp.float32)
    oh2 = (lane == i2).astype(jnp.float32)
    ssum = oh1 + oh2

    ri = lax.broadcasted_iota(jnp.int32, (BNG, BNG), 0)
    ci = lax.broadcasted_iota(jnp.int32, (BNG, BNG), 1)
    ltri = (ri > ci).astype(jnp.bfloat16)
    excl = jnp.dot(ltri, ssum.astype(jnp.bfloat16),
                   preferred_element_type=jnp.float32)

    @pl.when(nb == 0)
    def _():
        cnt_ref[...] = jnp.zeros_like(cnt_ref)

    base = cnt_ref[0:1, 0:E]
    g0 = excl + base
    r0 = jnp.sum(g0 * oh1, axis=-1, keepdims=True)
    r1 = jnp.sum(g0 * oh2, axis=-1, keepdims=True)

    blockcnt = jnp.sum(ssum, axis=0, keepdims=True)
    cnt_ref[0:1, 0:E] = base + blockcnt

    r_ref[...] = jnp.concatenate(
        [i1.astype(jnp.float32), i2.astype(jnp.float32), r0, r1, w1, w2,
         jnp.zeros_like(w1), jnp.zeros_like(w1)], axis=1)

    @pl.when(nb == NBG - 1)
    def _():
        total = base + blockcnt
        padded = jnp.floor((total + (BM - 1)) / BM) * BM
        ea = lax.broadcasted_iota(jnp.int32, (E, E), 0)
        eb = lax.broadcasted_iota(jnp.int32, (E, E), 1)
        ut = (ea < eb).astype(jnp.float32)
        start = jnp.dot(padded, ut,
                        preferred_element_type=jnp.float32)
        start_p = jnp.concatenate(
            [start, jnp.zeros((1, 128 - E), jnp.float32)], axis=1)
        ss_ref[...] = start_p

        blk = lax.broadcasted_iota(jnp.int32, (1, 256), 1) * BM
        acc = jnp.full((1, 256), -1, jnp.int32)
        for e in range(E):
            acc = acc + jnp.where(blk >= start[0, e].astype(jnp.int32), 1, 0)
        be_ref[...] = acc


def _gate_route(x, gate_W, gate_b):
    return pl.pallas_call(
        _gate_body,
        grid=(NBG,),
        in_specs=[
            pl.BlockSpec((BNG, D), lambda nb: (nb, 0)),
            pl.BlockSpec((D, E), lambda nb: (0, 0)),
            pl.BlockSpec((E,), lambda nb: (0,)),
        ],
        out_specs=[
            pl.BlockSpec((BNG, E), lambda nb: (nb, 0)),
            pl.BlockSpec((BNG, 8), lambda nb: (nb, 0)),
            pl.BlockSpec((1, 128), lambda nb: (0, 0)),
            pl.BlockSpec((1, 256), lambda nb: (0, 0)),
        ],
        out_shape=[
            jax.ShapeDtypeStruct((N, E), jnp.float32),
            jax.ShapeDtypeStruct((N, 8), jnp.float32),
            jax.ShapeDtypeStruct((1, 128), jnp.float32),
            jax.ShapeDtypeStruct((1, 256), jnp.int32),
        ],
        scratch_shapes=[pltpu.VMEM((8, 128), jnp.float32)],
        compiler_params=pltpu.CompilerParams(
            dimension_semantics=("arbitrary",),
        ),
    )(x, gate_W, gate_b)


def _pos_chunk(rv, ssv, t0):
    tok = (lax.iota(jnp.int32, L) + t0) * 8
    i1 = plsc.load_gather(rv, [tok])
    i2 = plsc.load_gather(rv, [tok + 1])
    r0 = plsc.load_gather(rv, [tok + 2])
    r1 = plsc.load_gather(rv, [tok + 3])
    s0 = plsc.load_gather(ssv, [i1.astype(jnp.int32)])
    s1 = plsc.load_gather(ssv, [i2.astype(jnp.int32)])
    pos0 = (s0 + r0).astype(jnp.int32)
    pos1 = (s1 + r1).astype(jnp.int32)
    return pos0, pos1


def _dispatch(r, ss, xb3):
    mesh = plsc.VectorSubcoreMesh(core_axis_name="c", subcore_axis_name="s")

    @functools.partial(
        pl.kernel, mesh=mesh,
        compiler_params=pltpu.CompilerParams(needs_layout_passes=False),
        out_type=jax.ShapeDtypeStruct((S, D // 2), jnp.int32),
        scratch_types=[
            pltpu.VMEM((TW * 8,), jnp.float32),
            pltpu.VMEM((128,), jnp.float32),
            pltpu.VMEM((CS, D), jnp.float32),
            pltpu.VMEM((CS, D), jnp.float32),
            pltpu.VMEM((CS, D // 2), jnp.int32),
            pltpu.VMEM((CS, D // 2), jnp.int32),
            pltpu.VMEM((CS, D // 2), jnp.int32),
            pltpu.VMEM((CS,), jnp.int32),
            pltpu.VMEM((CS,), jnp.int32),
            pltpu.VMEM((CS,), jnp.int32),
            pltpu.VMEM((CS,), jnp.int32),
            pltpu.VMEM((CS,), jnp.int32),
            pltpu.VMEM((CS,), jnp.int32),
            pltpu.SemaphoreType.DMA,
            pltpu.SemaphoreType.DMA,
            pltpu.SemaphoreType.DMA,
            pltpu.SemaphoreType.DMA,
            pltpu.SemaphoreType.DMA,
            pltpu.SemaphoreType.DMA,
        ],
    )
    def k(r_hbm, ss_hbm, x_hbm, xs_hbm, rv, ssv, xb0, xb1, pb0, pb1, pb2,
          ia0, ia1, ia2, ib0, ib1, ib2, gs0, gs1, gs2, ws0, ws1, ws2):
        wid = lax.axis_index("s") * NC + lax.axis_index("c")
        base = wid * TW
        pltpu.sync_copy(r_hbm.at[pl.ds(base * 8, TW * 8)], rv)
        pltpu.sync_copy(ss_hbm.at[0], ssv)

        xbufs = (xb0, xb1)
        pbufs = (pb0, pb1, pb2)
        idx0 = (ia0, ia1, ia2)
        idx1 = (ib0, ib1, ib2)
        gsem = (gs0, gs1, gs2)
        wsem = (ws0, ws1, ws2)

        gh = [None, None]
        sh = [None] * NCH
        gh[0] = pltpu.async_copy(
            x_hbm.at[pl.ds(base, CS)], xbufs[0], gsem[0])
        for c in range(NCH):
            gb = c % 2
            b = c % 3
            if c >= 2:
                sh[c - 2][0].wait()
                sh[c - 2][1].wait()
            if c + 1 < NCH:
                gh[1 - gb] = pltpu.async_copy(
                    x_hbm.at[pl.ds(base + (c + 1) * CS, CS)],
                    xbufs[1 - gb], gsem[1 - gb])
            gh[gb].wait()

            def _pack_row(j, xb=xbufs[gb], pb=pbufs[b]):
                for rr in range(CS):
                    a = xb[rr, pl.ds(j * L, L)]
                    bb = xb[rr, pl.ds(D // 2 + j * L, L)]
                    w = plsc.bitcast(
                        plsc.pack(a, bb, format=plsc.PackFormat.INTERLEAVED),
                        jnp.int32)
                    pb[rr, pl.ds(j * L, L)] = w
                return j

            lax.fori_loop(0, D // 2 // L, lambda j, _: (_pack_row(j), 0)[1],
                          0, unroll=False)
            for k2 in range(CS // L):
                p0, p1 = _pos_chunk(rv, ssv, c * CS + k2 * L)
                idx0[b][pl.ds(k2 * L, L)] = p0
                idx1[b][pl.ds(k2 * L, L)] = p1
            sh[c] = (
                pltpu.async_copy(pbufs[b], xs_hbm.at[idx0[b]], wsem[b]),
                pltpu.async_copy(pbufs[b], xs_hbm.at[idx1[b]], wsem[b]),
            )
        sh[NCH - 2][0].wait()
        sh[NCH - 2][1].wait()
        sh[NCH - 1][0].wait()
        sh[NCH - 1][1].wait()

    return k(r, ss, xb3)


def _mlp_body(be_ref, xs_ref, w1_ref, b1_ref, w2_ref, b2_ref, y_ref,
              w1c_ref):
    i = pl.program_id(0)
    changed = jnp.logical_or(
        i == 0, be_ref[i] != be_ref[jnp.maximum(i - 1, 0)])

    @pl.when(changed)
    def _():
        w1c_ref[...] = w1_ref[0].astype(jnp.bfloat16)

    w = xs_ref[...]
    xa = jax.lax.bitcast_convert_type(
        jnp.left_shift(w, 16), jnp.float32).astype(jnp.bfloat16)
    xb = jax.lax.bitcast_convert_type(
        jnp.bitwise_and(w, jnp.int32(-65536)), jnp.float32
    ).astype(jnp.bfloat16)
    h = (jnp.dot(xa, w1c_ref[0:D // 2],
                 preferred_element_type=jnp.float32)
         + jnp.dot(xb, w1c_ref[D // 2:D],
                   preferred_element_type=jnp.float32)) + b1_ref[0]
    h = jnp.maximum(h, 0.0).astype(jnp.bfloat16)
    y_ref[...] = jnp.dot(h, w2_ref[0],
                         preferred_element_type=jnp.float32) + b2_ref[0]


def _grouped_mlp(be, xs2, w1b, b1r, w2p, b2p):
    grid_spec = pltpu.PrefetchScalarGridSpec(
        num_scalar_prefetch=1,
        grid=(NBS,),
        in_specs=[
            pl.BlockSpec((BM, D // 2), lambda i, be: (i, 0)),
            pl.BlockSpec((1, D, H), lambda i, be: (be[i], 0, 0)),
            pl.BlockSpec((1, 1, H), lambda i, be: (be[i], 0, 0)),
            pl.BlockSpec((1, H, OP), lambda i, be: (be[i], 0, 0)),
            pl.BlockSpec((1, 1, OP), lambda i, be: (be[i], 0, 0)),
        ],
        out_specs=pl.BlockSpec((BM, OP), lambda i, be: (i, 0)),
        scratch_shapes=[pltpu.VMEM((D, H), jnp.bfloat16)],
    )
    return pl.pallas_call(
        _mlp_body,
        grid_spec=grid_spec,
        out_shape=jax.ShapeDtypeStruct((S, OP), jnp.float32),
        compiler_params=pltpu.CompilerParams(
            dimension_semantics=("arbitrary",),
        ),
    )(be, xs2, w1b, b1r, w2p, b2p)


def _combine(r, ss, y):
    mesh = plsc.VectorSubcoreMesh(core_axis_name="c", subcore_axis_name="s")

    @functools.partial(
        pl.kernel, mesh=mesh,
        compiler_params=pltpu.CompilerParams(needs_layout_passes=False),
        out_type=jax.ShapeDtypeStruct((N, OP), jnp.float32),
        scratch_types=[
            pltpu.VMEM((TW * 8,), jnp.float32),
            pltpu.VMEM((128,), jnp.float32),
            pltpu.VMEM((TW,), jnp.int32),
            pltpu.VMEM((TW,), jnp.int32),
            pltpu.VMEM((TW, OP), jnp.float32),
            pltpu.VMEM((TW, OP), jnp.float32),
            pltpu.VMEM((TW, OP), jnp.float32),
            pltpu.SemaphoreType.DMA,
            pltpu.SemaphoreType.DMA,
        ],
    )
    def k(r_hbm, ss_hbm, y_hbm, out_hbm, rv, ssv, ix0, ix1, y0, y1, ob,
          sem0, sem1):
        wid = lax.axis_index("s") * NC + lax.axis_index("c")
        base = wid * TW
        pltpu.sync_copy(r_hbm.at[pl.ds(base * 8, TW * 8)], rv)
        pltpu.sync_copy(ss_hbm.at[0], ssv)
        for k2 in range(TW // L):
            p0, p1 = _pos_chunk(rv, ssv, k2 * L)
            ix0[pl.ds(k2 * L, L)] = p0
            ix1[pl.ds(k2 * L, L)] = p1
        h0 = pltpu.async_copy(y_hbm.at[ix0], y0, sem0)
        h1 = pltpu.async_copy(y_hbm.at[ix1], y1, sem1)
        h0.wait()
        h1.wait()
        for t in range(TW):
            wv1 = plsc.load_gather(rv, [jnp.full((L,), t * 8 + 4, jnp.int32)])
            wv2 = plsc.load_gather(rv, [jnp.full((L,), t * 8 + 5, jnp.int32)])
            ob[t, pl.ds(0, L)] = (wv1 * y0[t, pl.ds(0, L)]
                                  + wv2 * y1[t, pl.ds(0, L)])
        pltpu.sync_copy(ob, out_hbm.at[pl.ds(base, TW)])

    return k(r, ss, y)


@jax.jit
def _moe(x, gate_W, gate_b, W1, b1, W2, b2):
    probs, r2d, ss, be2d = _gate_route(x, gate_W, gate_b)
    r = r2d.reshape(N * 8)
    be = be2d.reshape(-1)[:NBS]

    xs2 = _dispatch(r, ss, x)

    b1r = b1.reshape(E, 1, H)
    w2p = jnp.zeros((E, H, OP), jnp.bfloat16).at[:, :, :O].set(
        W2.astype(jnp.bfloat16))
    b2p = jnp.zeros((E, 1, OP), jnp.float32).at[:, 0, :O].set(b2)
    y = _grouped_mlp(be, xs2, W1, b1r, w2p, b2p)

    out16 = _combine(r, ss, y)
    return out16[:, :O], probs


def kernel(x, gate_W, gate_b, W1, b1, W2, b2):
    return _moe(x, gate_W, gate_b, W1, b1, W2, b2)
